# baseline (speedup 1.0000x reference)
import jax
import jax.numpy as jnp
from jax import lax
from jax.experimental import pallas as pl
from jax.experimental.pallas import tpu as pltpu

_N = 33554432
_COLS = 8192
_ROWS = _N // _COLS
_BLOCK_ROWS = 256
_CH_ROWS = 8
_CH_COLS = 1024
_CHUNKS = (_BLOCK_ROWS // _CH_ROWS) * (_COLS // _CH_COLS)


def _loss_chunk(x):
    cond = x <= -1.0
    safe = jnp.minimum(x, -1.0)
    return jnp.where(cond, -jnp.log(-safe), x + 1.0)


def _body(x_ref, o_ref, acc_ref):
    i = pl.program_id(0)

    def step(k, acc):
        r = (k // (_COLS // _CH_COLS)) * _CH_ROWS
        c = (k % (_COLS // _CH_COLS)) * _CH_COLS
        x = x_ref[pl.ds(r, _CH_ROWS), pl.ds(c, _CH_COLS)]
        return acc + _loss_chunk(x)

    acc = lax.fori_loop(
        0, _CHUNKS, step, jnp.zeros((_CH_ROWS, _CH_COLS), jnp.float32)
    )

    @pl.when(i == 0)
    def _():
        acc_ref[...] = jnp.zeros_like(acc_ref)

    acc_ref[...] += acc

    @pl.when(i == pl.num_programs(0) - 1)
    def _():
        o_ref[0] = jnp.sum(acc_ref[...]) / _N


def kernel(fx):
    x2d = fx.reshape(_ROWS, _COLS)
    out = pl.pallas_call(
        _body,
        grid=(_ROWS // _BLOCK_ROWS,),
        in_specs=[pl.BlockSpec((_BLOCK_ROWS, _COLS), lambda i: (i, 0))],
        out_specs=pl.BlockSpec(memory_space=pltpu.SMEM),
        out_shape=jax.ShapeDtypeStruct((1,), jnp.float32),
        scratch_shapes=[pltpu.VMEM((_CH_ROWS, _CH_COLS), jnp.float32)],
        compiler_params=pltpu.CompilerParams(
            dimension_semantics=("arbitrary",),
        ),
    )(x2d)
    return out[0]


# speedup vs baseline: 1.2655x; 1.2655x over previous
import jax
import jax.numpy as jnp
from jax import lax
from jax.experimental import pallas as pl
from jax.experimental.pallas import tpu as pltpu

_N = 33554432
_COLS = 8192
_ROWS = _N // _COLS
_BLOCK_ROWS = 256
_CH_ROWS = 8
_CH_COLS = 1024
_UNROLL = 4
_CHUNKS = (_BLOCK_ROWS // _CH_ROWS) * (_COLS // _CH_COLS)
_STEPS = _CHUNKS // _UNROLL


def _term(x):
    return jnp.maximum(x, -1.0) - jnp.log(jnp.maximum(-x, 1.0))


def _body(x_ref, o_ref, acc_ref):
    i = pl.program_id(0)
    ncol = _COLS // _CH_COLS

    def step(k, accs):
        base = k * _UNROLL
        new = []
        for u in range(_UNROLL):
            kk = base + u
            r = (kk // ncol) * _CH_ROWS
            c = (kk % ncol) * _CH_COLS
            x = x_ref[pl.ds(r, _CH_ROWS), pl.ds(c, _CH_COLS)]
            new.append(accs[u] + _term(x))
        return tuple(new)

    z = jnp.zeros((_CH_ROWS, _CH_COLS), jnp.float32)
    accs = lax.fori_loop(0, _STEPS, step, (z,) * _UNROLL)
    acc = (accs[0] + accs[1]) + (accs[2] + accs[3])

    @pl.when(i == 0)
    def _():
        acc_ref[...] = jnp.zeros_like(acc_ref)

    acc_ref[...] += acc

    @pl.when(i == pl.num_programs(0) - 1)
    def _():
        o_ref[0] = jnp.sum(acc_ref[...]) / _N + 1.0


def kernel(fx):
    x2d = fx.reshape(_ROWS, _COLS)
    out = pl.pallas_call(
        _body,
        grid=(_ROWS // _BLOCK_ROWS,),
        in_specs=[pl.BlockSpec((_BLOCK_ROWS, _COLS), lambda i: (i, 0))],
        out_specs=pl.BlockSpec(memory_space=pltpu.SMEM),
        out_shape=jax.ShapeDtypeStruct((1,), jnp.float32),
        scratch_shapes=[pltpu.VMEM((_CH_ROWS, _CH_COLS), jnp.float32)],
        compiler_params=pltpu.CompilerParams(
            dimension_semantics=("arbitrary",),
        ),
    )(x2d)
    return out[0]
